# baseline (speedup 1.0000x reference)
import functools

import jax
import jax.numpy as jnp
from jax import lax
from jax.experimental import pallas as pl
from jax.experimental.pallas import tpu as pltpu
from jax.experimental.pallas import tpu_sc as plsc

NC = 2
NS = 16
NW = NC * NS
LANES = 16
C = 128
ZR = 32
R_TC = 512


def _ceil_to(x, m):
  return (x + m - 1) // m * m


def _make_deg_kernel(e_pad, n_sp):
  ew = e_pad // NW
  n_iter = ew // C
  mesh = plsc.VectorSubcoreMesh(core_axis_name="c", subcore_axis_name="s")

  @functools.partial(
      pl.kernel,
      out_type=jax.ShapeDtypeStruct((2, NW, n_sp), jnp.float32),
      mesh=mesh,
      compiler_params=pltpu.CompilerParams(needs_layout_passes=False),
      scratch_types=[
          pltpu.VMEM((C,), jnp.int32),
          pltpu.VMEM((C,), jnp.int32),
          pltpu.VMEM((n_sp,), jnp.float32),
          pltpu.VMEM((n_sp,), jnp.float32),
      ],
  )
  def deg_kernel(src_hbm, dst_hbm, out_hbm, src_buf, dst_buf, hist_o, hist_i):
    c = lax.axis_index("c")
    s = lax.axis_index("s")
    wid = s * NC + c
    ones = jnp.ones((LANES,), jnp.float32)

    def zz(i, carry):
      hist_o[pl.ds(i * LANES, LANES)] = jnp.zeros((LANES,), jnp.float32)
      hist_i[pl.ds(i * LANES, LANES)] = jnp.zeros((LANES,), jnp.float32)
      return carry

    lax.fori_loop(0, n_sp // LANES, zz, 0)

    base0 = wid * ew

    def body(i, carry):
      base = base0 + i * C
      pltpu.sync_copy(src_hbm.at[pl.ds(base, C)], src_buf)
      pltpu.sync_copy(dst_hbm.at[pl.ds(base, C)], dst_buf)
      for k in range(C // LANES):
        si = src_buf[pl.ds(k * LANES, LANES)]
        di = dst_buf[pl.ds(k * LANES, LANES)]
        plsc.addupdate_scatter(hist_o, [si], ones)
        plsc.addupdate_scatter(hist_i, [di], ones)
      return carry

    lax.fori_loop(0, n_iter, body, 0)
    pltpu.sync_copy(hist_o, out_hbm.at[0, wid])
    pltpu.sync_copy(hist_i, out_hbm.at[1, wid])

  return deg_kernel


NBUF = 2
QC = 8


def _make_agg_kernel(e_pad, n_sp, d):
  rps = n_sp // NS
  ch_tot = e_pad // C
  k0 = ch_tot // NS
  assert k0 % (2 * QC) == 0
  mesh = plsc.VectorSubcoreMesh(core_axis_name="c", subcore_axis_name="s")

  @functools.partial(
      pl.kernel,
      out_type=jax.ShapeDtypeStruct((n_sp, d), jnp.float32),
      mesh=mesh,
      scratch_types=[
          pltpu.VMEM((2, QC, C), jnp.int32),
          pltpu.VMEM((2, QC, C), jnp.int32),
          pltpu.VMEM((NBUF, C, d), jnp.float32),
          pltpu.VMEM_SHARED((n_sp, d), jnp.float32),
          [pltpu.SemaphoreType.DMA] * NBUF,
          [pltpu.SemaphoreType.DMA] * 2,
          [pltpu.SemaphoreType.DMA] * 2,
      ],
  )
  def agg_kernel(u_hbm, src_hbm, dst_hbm, out_hbm, idx_s, idx_d, rows,
                 agg_sh, sems, semi_s, semi_d):
    c = lax.axis_index("c")
    s = lax.axis_index("s")

    def zero_and_run(chunk0, nblk):
      def fill(i, carry):
        for j in range(d // LANES):
          rows[0, i, pl.ds(j * LANES, LANES)] = jnp.zeros((LANES,), jnp.float32)
        return carry

      lax.fori_loop(0, C, fill, 0)

      def zcp(k, carry):
        pltpu.sync_copy(rows.at[0], agg_sh.at[pl.ds(s * rps + k * C, C)])
        return carry

      lax.fori_loop(0, rps // C, zcp, 0)
      plsc.subcore_barrier()
      def start_i(blk, par):
        base = chunk0 + blk * QC
        pltpu.async_copy(src_hbm.at[pl.ds(base, QC)], idx_s.at[par], semi_s[par])
        pltpu.async_copy(dst_hbm.at[pl.ds(base, QC)], idx_d.at[par], semi_d[par])

      def wait_i(par):
        pltpu.make_async_copy(src_hbm.at[pl.ds(0, QC)], idx_s.at[par], semi_s[par]).wait()
        pltpu.make_async_copy(dst_hbm.at[pl.ds(0, QC)], idx_d.at[par], semi_d[par]).wait()

      def start_g(par, jj, b):
        pltpu.async_copy(u_hbm.at[idx_s.at[par, jj]], rows.at[b], sems[b])

      def wait_g(par, jj, b):
        pltpu.make_async_copy(u_hbm.at[idx_s.at[par, jj]], rows.at[b], sems[b]).wait()

      start_i(0, 0)
      start_i(1, 1)

      def pairbody(blk2, carry):
        for par in range(2):
          blk = blk2 * 2 + par
          wait_i(par)
          start_g(par, 0, 0)
          for jj in range(QC):
            b = jj % NBUF
            if jj + 1 < QC:
              start_g(par, jj + 1, 1 - b)
            wait_g(par, jj, b)
            pltpu.sync_copy(rows.at[b], agg_sh.at[idx_d.at[par, jj]], add=True)

          @pl.when(blk + 2 < nblk)
          def _():
            start_i(blk + 2, par)

        return carry

      lax.fori_loop(0, nblk // 2, pairbody, 0)
      plsc.subcore_barrier()
      off = s * rps
      pltpu.sync_copy(agg_sh.at[pl.ds(off, rps)], out_hbm.at[pl.ds(off, rps)])

    @pl.when(c == 0)
    def _():
      zero_and_run(s * k0, k0 // QC)

  return agg_kernel


def _prep_tc(x_p, degot, degit, n):
  n_sp, d = x_p.shape
  nb = n_sp // R_TC

  def body(x_ref, do_ref, di_ref, u_ref, ns_ref, nd_ref):
    i = pl.program_id(0)
    rowid = lax.broadcasted_iota(jnp.int32, (R_TC, 1), 0) + i * R_TC
    valid = rowid < n
    deg_o = jnp.sum(do_ref[...], axis=1, keepdims=True)
    deg_i = jnp.sum(di_ref[...], axis=1, keepdims=True)
    ns = jnp.where(valid, lax.rsqrt(jnp.maximum(deg_o, 1.0)), 0.0)
    nd = jnp.where(valid, lax.rsqrt(jnp.maximum(deg_i, 1.0)), 0.0)
    ns_ref[...] = ns
    nd_ref[...] = nd
    u_ref[...] = x_ref[...] * ns

  vec = pl.BlockSpec((R_TC, 1), lambda i: (i, 0))
  mat = pl.BlockSpec((R_TC, d), lambda i: (i, 0))
  part = pl.BlockSpec((R_TC, NW), lambda i: (i, 0))
  return pl.pallas_call(
      body,
      grid=(nb,),
      in_specs=[mat, part, part],
      out_specs=[mat, vec, vec],
      out_shape=[
          jax.ShapeDtypeStruct((n_sp, d), jnp.float32),
          jax.ShapeDtypeStruct((n_sp, 1), jnp.float32),
          jax.ShapeDtypeStruct((n_sp, 1), jnp.float32),
      ],
  )(x_p, degot, degit)


def _layer_tc(a, nd, ns, w, b):
  n_sp, d = a.shape
  nb = n_sp // R_TC

  def body(a_ref, nd_ref, ns_ref, w_ref, b_ref, u_ref):
    t = a_ref[...] * nd_ref[...]
    h = jnp.dot(t, w_ref[...], preferred_element_type=jnp.float32) + b_ref[...]
    h = jnp.where(h >= 0, h, 0.1 * h)
    u_ref[...] = h * ns_ref[...]

  vec = pl.BlockSpec((R_TC, 1), lambda i: (i, 0))
  mat = pl.BlockSpec((R_TC, d), lambda i: (i, 0))
  full = pl.BlockSpec((d, d), lambda i: (0, 0))
  brow = pl.BlockSpec((1, d), lambda i: (0, 0))
  return pl.pallas_call(
      body,
      grid=(nb,),
      in_specs=[mat, vec, vec, full, brow],
      out_specs=mat,
      out_shape=jax.ShapeDtypeStruct((n_sp, d), jnp.float32),
  )(a, nd, ns, w, b.reshape(1, d))


def _final_tc(a, nd, w3, b3, w_out, b_out, n):
  n_sp, d = a.shape
  nb = n_sp // R_TC

  def body(a_ref, nd_ref, w3_ref, b3_ref, wo_ref, bo_ref, out_ref,
           acc_ref):
    i = pl.program_id(0)

    @pl.when(i == 0)
    def _():
      acc_ref[...] = jnp.zeros_like(acc_ref)

    t = a_ref[...] * nd_ref[...]
    acc_ref[...] += jnp.sum(t, axis=0, keepdims=True)

    @pl.when(i == nb - 1)
    def _():
      r = acc_ref[...] * (1.0 / n)
      h = jnp.dot(r, w3_ref[...], preferred_element_type=jnp.float32) + b3_ref[...]
      out_ref[...] = (
          jnp.dot(h, wo_ref[...], preferred_element_type=jnp.float32) + bo_ref[...]
      )

  vec = pl.BlockSpec((R_TC, 1), lambda i: (i, 0))
  mat = pl.BlockSpec((R_TC, d), lambda i: (i, 0))
  return pl.pallas_call(
      body,
      grid=(nb,),
      in_specs=[
          mat, vec,
          pl.BlockSpec((d, d), lambda i: (0, 0)),
          pl.BlockSpec((1, d), lambda i: (0, 0)),
          pl.BlockSpec((d, 1), lambda i: (0, 0)),
          pl.BlockSpec((1, 1), lambda i: (0, 0)),
      ],
      out_specs=pl.BlockSpec((1, 1), lambda i: (0, 0)),
      out_shape=jax.ShapeDtypeStruct((1, 1), jnp.float32),
      scratch_shapes=[pltpu.VMEM((1, d), jnp.float32)],
  )(a, nd, w3, b3.reshape(1, d), w_out, b_out.reshape(1, 1))


def kernel(in_feat, edge_index, W1, b1, W2, b2, W3, b3, W_out, b_out):
  n, d = in_feat.shape
  e = edge_index.shape[1]
  n_sp = _ceil_to(n + 1, NS * ZR)
  e_pad = _ceil_to(e, NW * C * 16)
  pad = e_pad - e

  src = edge_index[0].astype(jnp.int32)
  dst = edge_index[1].astype(jnp.int32)
  fill = jnp.full((pad,), n, jnp.int32)
  src_p = jnp.concatenate([src, fill])
  dst_p = jnp.concatenate([dst, fill])
  x_p = jnp.pad(in_feat, ((0, n_sp - n), (0, 0)))

  degs = _make_deg_kernel(e_pad, n_sp)(src_p, dst_p)
  degot = degs[0].T
  degit = degs[1].T

  u0, nsrc, ndst = _prep_tc(x_p, degot, degit, n)

  src3 = src_p.reshape(e_pad // C, C)
  dst3 = dst_p.reshape(e_pad // C, C)
  agg = _make_agg_kernel(e_pad, n_sp, d)
  a = agg(u0, src3, dst3)
  u1 = _layer_tc(a, ndst, nsrc, W1, b1)
  a = agg(u1, src3, dst3)
  u2 = _layer_tc(a, ndst, nsrc, W2, b2)
  a = agg(u2, src3, dst3)
  u3 = _layer_tc(a, ndst, nsrc, W2, b2)
  a = agg(u3, src3, dst3)
  return _final_tc(a, ndst, W3, b3, W_out, b_out, n)


# speedup vs baseline: 1.3177x; 1.3177x over previous
import functools

import jax
import jax.numpy as jnp
from jax import lax
from jax.experimental import pallas as pl
from jax.experimental.pallas import tpu as pltpu
from jax.experimental.pallas import tpu_sc as plsc

NC = 2
NS = 16
NW = NC * NS
LANES = 16
C = 128
ZR = 32
R_TC = 512


def _ceil_to(x, m):
  return (x + m - 1) // m * m


def _make_deg_kernel(e_pad, n_sp):
  ew = e_pad // NW
  n_iter = ew // C
  mesh = plsc.VectorSubcoreMesh(core_axis_name="c", subcore_axis_name="s")

  @functools.partial(
      pl.kernel,
      out_type=jax.ShapeDtypeStruct((2, NW, n_sp), jnp.float32),
      mesh=mesh,
      compiler_params=pltpu.CompilerParams(needs_layout_passes=False),
      scratch_types=[
          pltpu.VMEM((C,), jnp.int32),
          pltpu.VMEM((C,), jnp.int32),
          pltpu.VMEM((n_sp,), jnp.float32),
          pltpu.VMEM((n_sp,), jnp.float32),
      ],
  )
  def deg_kernel(src_hbm, dst_hbm, out_hbm, src_buf, dst_buf, hist_o, hist_i):
    c = lax.axis_index("c")
    s = lax.axis_index("s")
    wid = s * NC + c
    ones = jnp.ones((LANES,), jnp.float32)

    def zz(i, carry):
      hist_o[pl.ds(i * LANES, LANES)] = jnp.zeros((LANES,), jnp.float32)
      hist_i[pl.ds(i * LANES, LANES)] = jnp.zeros((LANES,), jnp.float32)
      return carry

    lax.fori_loop(0, n_sp // LANES, zz, 0)

    base0 = wid * ew

    def body(i, carry):
      base = base0 + i * C
      pltpu.sync_copy(src_hbm.at[pl.ds(base, C)], src_buf)
      pltpu.sync_copy(dst_hbm.at[pl.ds(base, C)], dst_buf)
      for k in range(C // LANES):
        si = src_buf[pl.ds(k * LANES, LANES)]
        di = dst_buf[pl.ds(k * LANES, LANES)]
        plsc.addupdate_scatter(hist_o, [si], ones)
        plsc.addupdate_scatter(hist_i, [di], ones)
      return carry

    lax.fori_loop(0, n_iter, body, 0)
    pltpu.sync_copy(hist_o, out_hbm.at[0, wid])
    pltpu.sync_copy(hist_i, out_hbm.at[1, wid])

  return deg_kernel


NBUF = 2
QC = 8
F_BIG = 0.9
BIG_CORE = 1


def _make_agg_kernel(e_pad, n_sp, d):
  rps = n_sp // NS
  ch_tot = e_pad // C
  per_sub = ch_tot // NS
  k_big = int(per_sub * F_BIG) // (2 * QC) * (2 * QC)
  k_sml = per_sub - k_big
  assert k_sml % (2 * QC) == 0 and k_big > 0 and k_sml > 0
  mesh = plsc.VectorSubcoreMesh(core_axis_name="c", subcore_axis_name="s")

  @functools.partial(
      pl.kernel,
      out_type=jax.ShapeDtypeStruct((NC, n_sp, d), jnp.float32),
      mesh=mesh,
      scratch_types=[
          pltpu.VMEM((2, QC, C), jnp.int32),
          pltpu.VMEM((2, QC, C), jnp.int32),
          pltpu.VMEM((NBUF, C, d), jnp.float32),
          pltpu.VMEM_SHARED((n_sp, d), jnp.float32),
          [pltpu.SemaphoreType.DMA] * NBUF,
          [pltpu.SemaphoreType.DMA] * 2,
          [pltpu.SemaphoreType.DMA] * 2,
      ],
  )
  def agg_kernel(u_hbm, src_hbm, dst_hbm, out_hbm, idx_s, idx_d, rows,
                 agg_sh, sems, semi_s, semi_d):
    c = lax.axis_index("c")
    s = lax.axis_index("s")

    def fill(i, carry):
      for j in range(d // LANES):
        rows[0, i, pl.ds(j * LANES, LANES)] = jnp.zeros((LANES,), jnp.float32)
      return carry

    lax.fori_loop(0, C, fill, 0)

    def zcp(k, carry):
      pltpu.sync_copy(rows.at[0], agg_sh.at[pl.ds(s * rps + k * C, C)])
      return carry

    lax.fori_loop(0, rps // C, zcp, 0)
    plsc.subcore_barrier()

    def run(chunk0, nblk):
      def start_i(blk, par):
        base = chunk0 + blk * QC
        pltpu.async_copy(src_hbm.at[pl.ds(base, QC)], idx_s.at[par], semi_s[par])
        pltpu.async_copy(dst_hbm.at[pl.ds(base, QC)], idx_d.at[par], semi_d[par])

      def wait_i(par):
        pltpu.make_async_copy(src_hbm.at[pl.ds(0, QC)], idx_s.at[par], semi_s[par]).wait()
        pltpu.make_async_copy(dst_hbm.at[pl.ds(0, QC)], idx_d.at[par], semi_d[par]).wait()

      def start_g(par, jj, b):
        pltpu.async_copy(u_hbm.at[idx_s.at[par, jj]], rows.at[b], sems[b])

      def wait_g(par, jj, b):
        pltpu.make_async_copy(u_hbm.at[idx_s.at[par, jj]], rows.at[b], sems[b]).wait()

      start_i(0, 0)
      start_i(1, 1)

      def pairbody(blk2, carry):
        for par in range(2):
          blk = blk2 * 2 + par
          wait_i(par)
          start_g(par, 0, 0)
          for jj in range(QC):
            b = jj % NBUF
            if jj + 1 < QC:
              start_g(par, jj + 1, 1 - b)
            wait_g(par, jj, b)
            pltpu.sync_copy(rows.at[b], agg_sh.at[idx_d.at[par, jj]], add=True)

          @pl.when(blk + 2 < nblk)
          def _():
            start_i(blk + 2, par)

        return carry

      lax.fori_loop(0, nblk // 2, pairbody, 0)

    @pl.when(c == BIG_CORE)
    def _():
      run(s * k_big, k_big // QC)

    @pl.when(c == 1 - BIG_CORE)
    def _():
      run(NS * k_big + s * k_sml, k_sml // QC)

    plsc.subcore_barrier()
    off = s * rps
    pltpu.sync_copy(agg_sh.at[pl.ds(off, rps)], out_hbm.at[c, pl.ds(off, rps)])

  return agg_kernel


def _prep_tc(x_p, degot, degit, n):
  n_sp, d = x_p.shape
  nb = n_sp // R_TC

  def body(x_ref, do_ref, di_ref, u_ref, ns_ref, nd_ref):
    i = pl.program_id(0)
    rowid = lax.broadcasted_iota(jnp.int32, (R_TC, 1), 0) + i * R_TC
    valid = rowid < n
    deg_o = jnp.sum(do_ref[...], axis=1, keepdims=True)
    deg_i = jnp.sum(di_ref[...], axis=1, keepdims=True)
    ns = jnp.where(valid, lax.rsqrt(jnp.maximum(deg_o, 1.0)), 0.0)
    nd = jnp.where(valid, lax.rsqrt(jnp.maximum(deg_i, 1.0)), 0.0)
    ns_ref[...] = ns
    nd_ref[...] = nd
    u_ref[...] = x_ref[...] * ns

  vec = pl.BlockSpec((R_TC, 1), lambda i: (i, 0))
  mat = pl.BlockSpec((R_TC, d), lambda i: (i, 0))
  part = pl.BlockSpec((R_TC, NW), lambda i: (i, 0))
  return pl.pallas_call(
      body,
      grid=(nb,),
      in_specs=[mat, part, part],
      out_specs=[mat, vec, vec],
      out_shape=[
          jax.ShapeDtypeStruct((n_sp, d), jnp.float32),
          jax.ShapeDtypeStruct((n_sp, 1), jnp.float32),
          jax.ShapeDtypeStruct((n_sp, 1), jnp.float32),
      ],
  )(x_p, degot, degit)


def _layer_tc(a0, a1, nd, ns, w, b):
  n_sp, d = a0.shape
  nb = n_sp // R_TC

  def body(a0_ref, a1_ref, nd_ref, ns_ref, w_ref, b_ref, u_ref):
    t = (a0_ref[...] + a1_ref[...]) * nd_ref[...]
    h = jnp.dot(t, w_ref[...], preferred_element_type=jnp.float32) + b_ref[...]
    h = jnp.where(h >= 0, h, 0.1 * h)
    u_ref[...] = h * ns_ref[...]

  vec = pl.BlockSpec((R_TC, 1), lambda i: (i, 0))
  mat = pl.BlockSpec((R_TC, d), lambda i: (i, 0))
  full = pl.BlockSpec((d, d), lambda i: (0, 0))
  brow = pl.BlockSpec((1, d), lambda i: (0, 0))
  return pl.pallas_call(
      body,
      grid=(nb,),
      in_specs=[mat, mat, vec, vec, full, brow],
      out_specs=mat,
      out_shape=jax.ShapeDtypeStruct((n_sp, d), jnp.float32),
  )(a0, a1, nd, ns, w, b.reshape(1, d))


def _final_tc(a0, a1, nd, w3, b3, w_out, b_out, n):
  n_sp, d = a0.shape
  nb = n_sp // R_TC

  def body(a0_ref, a1_ref, nd_ref, w3_ref, b3_ref, wo_ref, bo_ref, out_ref,
           acc_ref):
    i = pl.program_id(0)

    @pl.when(i == 0)
    def _():
      acc_ref[...] = jnp.zeros_like(acc_ref)

    t = (a0_ref[...] + a1_ref[...]) * nd_ref[...]
    acc_ref[...] += jnp.sum(t, axis=0, keepdims=True)

    @pl.when(i == nb - 1)
    def _():
      r = acc_ref[...] * (1.0 / n)
      h = jnp.dot(r, w3_ref[...], preferred_element_type=jnp.float32) + b3_ref[...]
      out_ref[...] = (
          jnp.dot(h, wo_ref[...], preferred_element_type=jnp.float32) + bo_ref[...]
      )

  vec = pl.BlockSpec((R_TC, 1), lambda i: (i, 0))
  mat = pl.BlockSpec((R_TC, d), lambda i: (i, 0))
  return pl.pallas_call(
      body,
      grid=(nb,),
      in_specs=[
          mat, mat, vec,
          pl.BlockSpec((d, d), lambda i: (0, 0)),
          pl.BlockSpec((1, d), lambda i: (0, 0)),
          pl.BlockSpec((d, 1), lambda i: (0, 0)),
          pl.BlockSpec((1, 1), lambda i: (0, 0)),
      ],
      out_specs=pl.BlockSpec((1, 1), lambda i: (0, 0)),
      out_shape=jax.ShapeDtypeStruct((1, 1), jnp.float32),
      scratch_shapes=[pltpu.VMEM((1, d), jnp.float32)],
  )(a0, a1, nd, w3, b3.reshape(1, d), w_out, b_out.reshape(1, 1))


def kernel(in_feat, edge_index, W1, b1, W2, b2, W3, b3, W_out, b_out):
  n, d = in_feat.shape
  e = edge_index.shape[1]
  n_sp = _ceil_to(n + 1, NS * ZR)
  e_pad = _ceil_to(e, NW * C * 16)
  pad = e_pad - e

  src = edge_index[0].astype(jnp.int32)
  dst = edge_index[1].astype(jnp.int32)
  fill = jnp.full((pad,), n, jnp.int32)
  src_p = jnp.concatenate([src, fill])
  dst_p = jnp.concatenate([dst, fill])
  x_p = jnp.pad(in_feat, ((0, n_sp - n), (0, 0)))

  degs = _make_deg_kernel(e_pad, n_sp)(src_p, dst_p)
  degot = degs[0].T
  degit = degs[1].T

  u0, nsrc, ndst = _prep_tc(x_p, degot, degit, n)

  src3 = src_p.reshape(e_pad // C, C)
  dst3 = dst_p.reshape(e_pad // C, C)
  agg = _make_agg_kernel(e_pad, n_sp, d)
  a = agg(u0, src3, dst3)
  u1 = _layer_tc(a[0], a[1], ndst, nsrc, W1, b1)
  a = agg(u1, src3, dst3)
  u2 = _layer_tc(a[0], a[1], ndst, nsrc, W2, b2)
  a = agg(u2, src3, dst3)
  u3 = _layer_tc(a[0], a[1], ndst, nsrc, W2, b2)
  a = agg(u3, src3, dst3)
  return _final_tc(a[0], a[1], ndst, W3, b3, W_out, b_out, n)


# speedup vs baseline: 1.3307x; 1.0098x over previous
import functools

import jax
import jax.numpy as jnp
from jax import lax
from jax.experimental import pallas as pl
from jax.experimental.pallas import tpu as pltpu
from jax.experimental.pallas import tpu_sc as plsc

NC = 2
NS = 16
NW = NC * NS
LANES = 16
C = 128
ZR = 32
R_TC = 512


def _ceil_to(x, m):
  return (x + m - 1) // m * m


def _make_deg_kernel(e_pad, n_sp):
  ew = e_pad // NW
  n_iter = ew // C
  mesh = plsc.VectorSubcoreMesh(core_axis_name="c", subcore_axis_name="s")

  @functools.partial(
      pl.kernel,
      out_type=jax.ShapeDtypeStruct((2, NW, n_sp), jnp.float32),
      mesh=mesh,
      compiler_params=pltpu.CompilerParams(needs_layout_passes=False),
      scratch_types=[
          pltpu.VMEM((C,), jnp.int32),
          pltpu.VMEM((C,), jnp.int32),
          pltpu.VMEM((n_sp,), jnp.float32),
          pltpu.VMEM((n_sp,), jnp.float32),
      ],
  )
  def deg_kernel(src_hbm, dst_hbm, out_hbm, src_buf, dst_buf, hist_o, hist_i):
    c = lax.axis_index("c")
    s = lax.axis_index("s")
    wid = s * NC + c
    ones = jnp.ones((LANES,), jnp.float32)

    def zz(i, carry):
      hist_o[pl.ds(i * LANES, LANES)] = jnp.zeros((LANES,), jnp.float32)
      hist_i[pl.ds(i * LANES, LANES)] = jnp.zeros((LANES,), jnp.float32)
      return carry

    lax.fori_loop(0, n_sp // LANES, zz, 0)

    base0 = wid * ew

    def body(i, carry):
      base = base0 + i * C
      pltpu.sync_copy(src_hbm.at[pl.ds(base, C)], src_buf)
      pltpu.sync_copy(dst_hbm.at[pl.ds(base, C)], dst_buf)
      for k in range(C // LANES):
        si = src_buf[pl.ds(k * LANES, LANES)]
        di = dst_buf[pl.ds(k * LANES, LANES)]
        plsc.addupdate_scatter(hist_o, [si], ones)
        plsc.addupdate_scatter(hist_i, [di], ones)
      return carry

    lax.fori_loop(0, n_iter, body, 0)
    pltpu.sync_copy(hist_o, out_hbm.at[0, wid])
    pltpu.sync_copy(hist_i, out_hbm.at[1, wid])

  return deg_kernel


NBUF = 2
QC = 4
F_BIG = 0.95
BIG_CORE = 1


def _make_agg_kernel(e_pad, n_sp, d):
  rps = n_sp // NS
  ch_tot = e_pad // C
  per_sub = ch_tot // NS
  k_big = int(per_sub * F_BIG) // (2 * QC) * (2 * QC)
  k_sml = per_sub - k_big
  assert k_sml % (2 * QC) == 0 and k_big > 0 and k_sml > 0
  mesh = plsc.VectorSubcoreMesh(core_axis_name="c", subcore_axis_name="s")

  @functools.partial(
      pl.kernel,
      out_type=jax.ShapeDtypeStruct((NC, n_sp, d), jnp.float32),
      mesh=mesh,
      scratch_types=[
          pltpu.VMEM((2, QC, C), jnp.int32),
          pltpu.VMEM((2, QC, C), jnp.int32),
          pltpu.VMEM((NBUF, C, d), jnp.float32),
          pltpu.VMEM_SHARED((n_sp, d), jnp.float32),
          [pltpu.SemaphoreType.DMA] * NBUF,
          [pltpu.SemaphoreType.DMA] * 2,
          [pltpu.SemaphoreType.DMA] * 2,
      ],
  )
  def agg_kernel(u_hbm, src_hbm, dst_hbm, out_hbm, idx_s, idx_d, rows,
                 agg_sh, sems, semi_s, semi_d):
    c = lax.axis_index("c")
    s = lax.axis_index("s")

    def fill(i, carry):
      for j in range(d // LANES):
        rows[0, i, pl.ds(j * LANES, LANES)] = jnp.zeros((LANES,), jnp.float32)
      return carry

    lax.fori_loop(0, C, fill, 0)

    def zcp(k, carry):
      pltpu.sync_copy(rows.at[0], agg_sh.at[pl.ds(s * rps + k * C, C)])
      return carry

    lax.fori_loop(0, rps // C, zcp, 0)
    plsc.subcore_barrier()

    def run(chunk0, nblk):
      def start_i(blk, par):
        base = chunk0 + blk * QC
        pltpu.async_copy(src_hbm.at[pl.ds(base, QC)], idx_s.at[par], semi_s[par])
        pltpu.async_copy(dst_hbm.at[pl.ds(base, QC)], idx_d.at[par], semi_d[par])

      def wait_i(par):
        pltpu.make_async_copy(src_hbm.at[pl.ds(0, QC)], idx_s.at[par], semi_s[par]).wait()
        pltpu.make_async_copy(dst_hbm.at[pl.ds(0, QC)], idx_d.at[par], semi_d[par]).wait()

      def start_g(par, jj, b):
        pltpu.async_copy(u_hbm.at[idx_s.at[par, jj]], rows.at[b], sems[b])

      def wait_g(par, jj, b):
        pltpu.make_async_copy(u_hbm.at[idx_s.at[par, jj]], rows.at[b], sems[b]).wait()

      start_i(0, 0)
      start_i(1, 1)

      def pairbody(blk2, carry):
        for par in range(2):
          blk = blk2 * 2 + par
          wait_i(par)
          start_g(par, 0, 0)
          for jj in range(QC):
            b = jj % NBUF
            if jj + 1 < QC:
              start_g(par, jj + 1, 1 - b)
            wait_g(par, jj, b)
            pltpu.sync_copy(rows.at[b], agg_sh.at[idx_d.at[par, jj]], add=True)

          @pl.when(blk + 2 < nblk)
          def _():
            start_i(blk + 2, par)

        return carry

      lax.fori_loop(0, nblk // 2, pairbody, 0)

    @pl.when(c == BIG_CORE)
    def _():
      run(s * k_big, k_big // QC)

    @pl.when(c == 1 - BIG_CORE)
    def _():
      run(NS * k_big + s * k_sml, k_sml // QC)

    plsc.subcore_barrier()
    off = s * rps
    pltpu.sync_copy(agg_sh.at[pl.ds(off, rps)], out_hbm.at[c, pl.ds(off, rps)])

  return agg_kernel


def _prep_tc(x_p, degot, degit, n):
  n_sp, d = x_p.shape
  nb = n_sp // R_TC

  def body(x_ref, do_ref, di_ref, u_ref, ns_ref, nd_ref):
    i = pl.program_id(0)
    rowid = lax.broadcasted_iota(jnp.int32, (R_TC, 1), 0) + i * R_TC
    valid = rowid < n
    deg_o = jnp.sum(do_ref[...], axis=1, keepdims=True)
    deg_i = jnp.sum(di_ref[...], axis=1, keepdims=True)
    ns = jnp.where(valid, lax.rsqrt(jnp.maximum(deg_o, 1.0)), 0.0)
    nd = jnp.where(valid, lax.rsqrt(jnp.maximum(deg_i, 1.0)), 0.0)
    ns_ref[...] = ns
    nd_ref[...] = nd
    u_ref[...] = x_ref[...] * ns

  vec = pl.BlockSpec((R_TC, 1), lambda i: (i, 0))
  mat = pl.BlockSpec((R_TC, d), lambda i: (i, 0))
  part = pl.BlockSpec((R_TC, NW), lambda i: (i, 0))
  return pl.pallas_call(
      body,
      grid=(nb,),
      in_specs=[mat, part, part],
      out_specs=[mat, vec, vec],
      out_shape=[
          jax.ShapeDtypeStruct((n_sp, d), jnp.float32),
          jax.ShapeDtypeStruct((n_sp, 1), jnp.float32),
          jax.ShapeDtypeStruct((n_sp, 1), jnp.float32),
      ],
  )(x_p, degot, degit)


def _layer_tc(a0, a1, nd, ns, w, b):
  n_sp, d = a0.shape
  nb = n_sp // R_TC

  def body(a0_ref, a1_ref, nd_ref, ns_ref, w_ref, b_ref, u_ref):
    t = (a0_ref[...] + a1_ref[...]) * nd_ref[...]
    h = jnp.dot(t, w_ref[...], preferred_element_type=jnp.float32) + b_ref[...]
    h = jnp.where(h >= 0, h, 0.1 * h)
    u_ref[...] = h * ns_ref[...]

  vec = pl.BlockSpec((R_TC, 1), lambda i: (i, 0))
  mat = pl.BlockSpec((R_TC, d), lambda i: (i, 0))
  full = pl.BlockSpec((d, d), lambda i: (0, 0))
  brow = pl.BlockSpec((1, d), lambda i: (0, 0))
  return pl.pallas_call(
      body,
      grid=(nb,),
      in_specs=[mat, mat, vec, vec, full, brow],
      out_specs=mat,
      out_shape=jax.ShapeDtypeStruct((n_sp, d), jnp.float32),
  )(a0, a1, nd, ns, w, b.reshape(1, d))


def _final_tc(a0, a1, nd, w3, b3, w_out, b_out, n):
  n_sp, d = a0.shape
  nb = n_sp // R_TC

  def body(a0_ref, a1_ref, nd_ref, w3_ref, b3_ref, wo_ref, bo_ref, out_ref,
           acc_ref):
    i = pl.program_id(0)

    @pl.when(i == 0)
    def _():
      acc_ref[...] = jnp.zeros_like(acc_ref)

    t = (a0_ref[...] + a1_ref[...]) * nd_ref[...]
    acc_ref[...] += jnp.sum(t, axis=0, keepdims=True)

    @pl.when(i == nb - 1)
    def _():
      r = acc_ref[...] * (1.0 / n)
      h = jnp.dot(r, w3_ref[...], preferred_element_type=jnp.float32) + b3_ref[...]
      out_ref[...] = (
          jnp.dot(h, wo_ref[...], preferred_element_type=jnp.float32) + bo_ref[...]
      )

  vec = pl.BlockSpec((R_TC, 1), lambda i: (i, 0))
  mat = pl.BlockSpec((R_TC, d), lambda i: (i, 0))
  return pl.pallas_call(
      body,
      grid=(nb,),
      in_specs=[
          mat, mat, vec,
          pl.BlockSpec((d, d), lambda i: (0, 0)),
          pl.BlockSpec((1, d), lambda i: (0, 0)),
          pl.BlockSpec((d, 1), lambda i: (0, 0)),
          pl.BlockSpec((1, 1), lambda i: (0, 0)),
      ],
      out_specs=pl.BlockSpec((1, 1), lambda i: (0, 0)),
      out_shape=jax.ShapeDtypeStruct((1, 1), jnp.float32),
      scratch_shapes=[pltpu.VMEM((1, d), jnp.float32)],
  )(a0, a1, nd, w3, b3.reshape(1, d), w_out, b_out.reshape(1, 1))


def kernel(in_feat, edge_index, W1, b1, W2, b2, W3, b3, W_out, b_out):
  n, d = in_feat.shape
  e = edge_index.shape[1]
  n_sp = _ceil_to(n + 1, NS * ZR)
  e_pad = _ceil_to(e, NW * C * 16)
  pad = e_pad - e

  src = edge_index[0].astype(jnp.int32)
  dst = edge_index[1].astype(jnp.int32)
  fill = jnp.full((pad,), n, jnp.int32)
  src_p = jnp.concatenate([src, fill])
  dst_p = jnp.concatenate([dst, fill])
  x_p = jnp.pad(in_feat, ((0, n_sp - n), (0, 0)))

  degs = _make_deg_kernel(e_pad, n_sp)(src_p, dst_p)
  degot = degs[0].T
  degit = degs[1].T

  u0, nsrc, ndst = _prep_tc(x_p, degot, degit, n)

  src3 = src_p.reshape(e_pad // C, C)
  dst3 = dst_p.reshape(e_pad // C, C)
  agg = _make_agg_kernel(e_pad, n_sp, d)
  a = agg(u0, src3, dst3)
  u1 = _layer_tc(a[0], a[1], ndst, nsrc, W1, b1)
  a = agg(u1, src3, dst3)
  u2 = _layer_tc(a[0], a[1], ndst, nsrc, W2, b2)
  a = agg(u2, src3, dst3)
  u3 = _layer_tc(a[0], a[1], ndst, nsrc, W2, b2)
  a = agg(u3, src3, dst3)
  return _final_tc(a[0], a[1], ndst, W3, b3, W_out, b_out, n)
